# baseline (speedup 1.0000x reference)
import functools

import jax
import jax.numpy as jnp
from jax import lax
from jax.experimental import pallas as pl
from jax.experimental.pallas import tpu as pltpu
from jax.experimental.pallas import tpu_sc as plsc

B = 16384
S = 6
D = 64
NW = 1000000
NC = 2
NS = 16
NSUB = NC * NS
BPT = (2 * B) // NSUB
CB = 64
NCHUNK = BPT // CB
VROWS = CB * S

TBLK = 32768
HALF = TBLK // 2

_GDN = lax.GatherDimensionNumbers(
    offset_dims=(), collapsed_slice_dims=(0,), start_index_map=(0,))


def _rot16(x, idx):
    return lax.gather(x, idx[:, None], dimension_numbers=_GDN,
                      slice_sizes=(1,),
                      mode=lax.GatherScatterMode.PROMISE_IN_BOUNDS)


def _sc_scores_kernel(word_hbm, doc_hbm, label_hbm, uq_hbm, vq_hbm,
                      vp_hbm, scores_hbm, u_idx, v_idx, rows_u, rows_v,
                      scores_v, v_par, sem):
    wid = lax.axis_index("s") * NC + lax.axis_index("c")
    br = wid % 2
    t = wid // 2
    lane = lax.iota(jnp.int32, 16)
    rots = [(lane + (1 << k)) & 15 for k in range(4)]

    def chunk_body(c, carry):
        b0 = br * B + t * BPT + c * CB
        pltpu.sync_copy(uq_hbm.at[pl.ds(pl.multiple_of(b0, CB), CB)], u_idx)
        pltpu.sync_copy(
            vq_hbm.at[pl.ds(pl.multiple_of(b0 * S, VROWS), VROWS)], v_idx)
        pltpu.sync_copy(
            vp_hbm.at[pl.ds(pl.multiple_of(b0 * S, VROWS), VROWS)], v_par)

        @pl.when(br == 0)
        def _():
            pltpu.async_copy(doc_hbm.at[u_idx], rows_u, sem).wait()

        @pl.when(br == 1)
        def _():
            pltpu.async_copy(label_hbm.at[u_idx], rows_u, sem).wait()

        for k in range(3):
            pltpu.async_copy(word_hbm.at[v_idx.at[pl.ds(k * 128, 128)]],
                             rows_v.at[pl.ds(k * 128, 128)], sem).wait()

        def blk_body(i, carry2):
            accs = [jnp.zeros((16,), jnp.float32) for _ in range(3)]
            pv = [v_par[pl.ds(i * 48 + 16 * tt, 16)] for tt in range(3)]
            for bb in range(8):
                b_loc = i * 8 + bb
                u = [rows_u[b_loc, pl.ds(16 * k, 16)] for k in range(4)]
                for s in range(S):
                    r = b_loc * S + s
                    j = bb * S + s
                    alo = u[0] * rows_v[r, pl.ds(0, 16)]
                    ahi = u[0] * rows_v[r, pl.ds(64, 16)]
                    alo += u[1] * rows_v[r, pl.ds(16, 16)]
                    ahi += u[1] * rows_v[r, pl.ds(80, 16)]
                    alo += u[2] * rows_v[r, pl.ds(32, 16)]
                    ahi += u[2] * rows_v[r, pl.ds(96, 16)]
                    alo += u[3] * rows_v[r, pl.ds(48, 16)]
                    ahi += u[3] * rows_v[r, pl.ds(112, 16)]
                    parf = _rot16(pv[j // 16],
                                  jnp.full((16,), j % 16, jnp.int32))
                    acc = alo + parf * (ahi - alo)
                    for idx in rots:
                        acc = acc + _rot16(acc, idx)
                    accs[j // 16] = jnp.where(lane == (j % 16), acc,
                                              accs[j // 16])
            for tt in range(3):
                scores_v[pl.ds(i * 48 + tt * 16, 16)] = accs[tt]
            return carry2

        lax.fori_loop(0, CB // 8, blk_body, 0)
        pltpu.sync_copy(
            scores_v,
            scores_hbm.at[pl.ds(pl.multiple_of(b0 * S, VROWS), VROWS)])
        return carry

    lax.fori_loop(0, NCHUNK, chunk_body, 0)


def _sc_scores(word_pk, doc_pk, label_pk, u_q, v_q, v_p):
    k = functools.partial(
        pl.kernel,
        mesh=plsc.VectorSubcoreMesh(core_axis_name="c", subcore_axis_name="s"),
        out_type=jax.ShapeDtypeStruct((2 * B * S,), jnp.float32),
        scratch_types=[
            pltpu.VMEM((CB,), jnp.int32),
            pltpu.VMEM((VROWS,), jnp.int32),
            pltpu.VMEM((CB, 2 * D), jnp.float32),
            pltpu.VMEM((VROWS, 2 * D), jnp.float32),
            pltpu.VMEM((VROWS,), jnp.float32),
            pltpu.VMEM((VROWS,), jnp.float32),
            pltpu.SemaphoreType.DMA,
        ],
        compiler_params=pltpu.CompilerParams(use_tc_tiling_on_sc=False),
    )(_sc_scores_kernel)
    return k(word_pk, doc_pk, label_pk, u_q, v_q, v_p)


def _tr_kernel(src_ref, dst_ref):
    x = src_ref[...]
    dst_ref[:, 0:D] = x[:, 0:HALF].T
    dst_ref[:, D:2 * D] = x[:, HALF:TBLK].T


def _transpose_pack(table_t, n_rows):
    grid = (n_rows + TBLK - 1) // TBLK
    return pl.pallas_call(
        _tr_kernel,
        grid=(grid,),
        in_specs=[pl.BlockSpec((D, TBLK), lambda i: (0, i))],
        out_specs=pl.BlockSpec((HALF, 2 * D), lambda i: (i, 0)),
        out_shape=jax.ShapeDtypeStruct((grid * HALF, 2 * D), jnp.float32),
        compiler_params=pltpu.CompilerParams(
            vmem_limit_bytes=100 * 1024 * 1024),
    )(table_t)


def _pack_ids(ids):
    return ((ids >> 15) << 14) | (ids & (HALF - 1)), (ids >> 14) & 1


def _tr_dup_kernel(src_ref, dst_ref):
    x = src_ref[...].T
    dst_ref[:, 0:D] = x
    dst_ref[:, D:2 * D] = x


def _transpose_dup(table_t, n_rows):
    grid = (n_rows + TBLK - 1) // TBLK
    return pl.pallas_call(
        _tr_dup_kernel,
        grid=(grid,),
        in_specs=[pl.BlockSpec((D, TBLK), lambda i: (0, i))],
        out_specs=pl.BlockSpec((TBLK, 2 * D), lambda i: (i, 0)),
        out_shape=jax.ShapeDtypeStruct((grid * TBLK, 2 * D), jnp.float32),
        compiler_params=pltpu.CompilerParams(
            vmem_limit_bytes=100 * 1024 * 1024),
    )(table_t)


def _loss_kernel(scores_ref, out_ref):
    x = scores_ref[...]
    r = lax.broadcasted_iota(jnp.int32, x.shape, 0)
    c = lax.broadcasted_iota(jnp.int32, x.shape, 1)
    s = (r * 128 + c) % S
    t = jnp.where(s == 0, -x, x)
    sp = jnp.maximum(t, 0.0) + jnp.log1p(jnp.exp(-jnp.abs(t)))
    out_ref[0, 0] = jnp.sum(sp) * (1.0 / B)


def kernel(dt, lt, word_embed, doc_embed, label_embed, noise_a, noise_b):
    u_ids = jnp.concatenate([dt[:, 0], lt[:, 0]]).astype(jnp.int32)
    v_ids = jnp.concatenate([
        jnp.concatenate([dt[:, 1:2], noise_a], axis=1),
        jnp.concatenate([lt[:, 1:2], noise_b], axis=1)], axis=0)
    v_ids = v_ids.astype(jnp.int32).reshape(2 * B * S)
    v_q, v_p = _pack_ids(v_ids)
    v_p = v_p.astype(jnp.float32)
    word_pk = _transpose_pack(word_embed.T, 1000000)
    doc_pk = _transpose_dup(doc_embed.T, 100000)
    label_pk = _transpose_dup(label_embed.T, 1000)
    scores = _sc_scores(word_pk, doc_pk, label_pk, u_ids, v_q, v_p)
    loss2d = pl.pallas_call(
        _loss_kernel,
        out_shape=jax.ShapeDtypeStruct((1, 1), jnp.float32),
        out_specs=pl.BlockSpec(memory_space=pltpu.SMEM),
    )(scores.reshape(2 * B * S // 128, 128))
    return loss2d.reshape(())


# speedup vs baseline: 1.1810x; 1.1810x over previous
import functools

import jax
import jax.numpy as jnp
from jax import lax
from jax.experimental import pallas as pl
from jax.experimental.pallas import tpu as pltpu
from jax.experimental.pallas import tpu_sc as plsc

B = 16384
S = 6
D = 64
NW = 1000000
NC = 2
NS = 16
NSUB = NC * NS
BPT = (2 * B) // NSUB
CB = 64
NCHUNK = BPT // CB
VROWS = CB * S

TBLK = 32768
HALF = TBLK // 2

_GDN = lax.GatherDimensionNumbers(
    offset_dims=(), collapsed_slice_dims=(0,), start_index_map=(0,))


def _rot16(x, idx):
    return lax.gather(x, idx[:, None], dimension_numbers=_GDN,
                      slice_sizes=(1,),
                      mode=lax.GatherScatterMode.PROMISE_IN_BOUNDS)


def _sc_scores_kernel(word_hbm, doc_hbm, label_hbm, uq_hbm, vq_hbm,
                      vp_hbm, scores_hbm,
                      u_idx0, v_idx0, vpar0, rows_u0, rows_v0,
                      u_idx1, v_idx1, vpar1, rows_u1, rows_v1,
                      scores_v, sem0, sem1):
    wid = lax.axis_index("s") * NC + lax.axis_index("c")
    br = wid % 2
    t = wid // 2
    lane = lax.iota(jnp.int32, 16)
    rots = [(lane + (1 << k)) & 15 for k in range(4)]
    bufs = [(u_idx0, v_idx0, vpar0, rows_u0, rows_v0, sem0),
            (u_idx1, v_idx1, vpar1, rows_u1, rows_v1, sem1)]

    def stage_issue(c, buf):
        u_idx, v_idx, v_par, rows_u, rows_v, sem = buf
        b0 = br * B + t * BPT + c * CB
        pltpu.sync_copy(uq_hbm.at[pl.ds(pl.multiple_of(b0, CB), CB)], u_idx)
        pltpu.sync_copy(
            vq_hbm.at[pl.ds(pl.multiple_of(b0 * S, VROWS), VROWS)], v_idx)
        pltpu.sync_copy(
            vp_hbm.at[pl.ds(pl.multiple_of(b0 * S, VROWS), VROWS)], v_par)

        @pl.when(br == 0)
        def _():
            pltpu.async_copy(doc_hbm.at[u_idx], rows_u, sem)

        @pl.when(br == 1)
        def _():
            pltpu.async_copy(label_hbm.at[u_idx], rows_u, sem)

        for k in range(3):
            pltpu.async_copy(word_hbm.at[v_idx.at[pl.ds(k * 128, 128)]],
                             rows_v.at[pl.ds(k * 128, 128)], sem)

    def wait_buf(buf):
        u_idx, v_idx, v_par, rows_u, rows_v, sem = buf
        pltpu.make_async_copy(doc_hbm.at[u_idx], rows_u, sem).wait()
        for k in range(3):
            pltpu.make_async_copy(word_hbm.at[v_idx.at[pl.ds(k * 128, 128)]],
                                  rows_v.at[pl.ds(k * 128, 128)], sem).wait()

    def compute(c, buf):
        u_idx, v_idx, v_par, rows_u, rows_v, sem = buf
        b0 = br * B + t * BPT + c * CB

        def blk_body(i, carry2):
            accs = [jnp.zeros((16,), jnp.float32) for _ in range(3)]
            pv = [v_par[pl.ds(i * 48 + 16 * tt, 16)] for tt in range(3)]
            for bb in range(8):
                b_loc = i * 8 + bb
                u = [rows_u[b_loc, pl.ds(16 * k, 16)] for k in range(4)]
                for s in range(S):
                    r = b_loc * S + s
                    j = bb * S + s
                    alo = u[0] * rows_v[r, pl.ds(0, 16)]
                    ahi = u[0] * rows_v[r, pl.ds(64, 16)]
                    alo += u[1] * rows_v[r, pl.ds(16, 16)]
                    ahi += u[1] * rows_v[r, pl.ds(80, 16)]
                    alo += u[2] * rows_v[r, pl.ds(32, 16)]
                    ahi += u[2] * rows_v[r, pl.ds(96, 16)]
                    alo += u[3] * rows_v[r, pl.ds(48, 16)]
                    ahi += u[3] * rows_v[r, pl.ds(112, 16)]
                    parf = _rot16(pv[j // 16],
                                  jnp.full((16,), j % 16, jnp.int32))
                    acc = alo + parf * (ahi - alo)
                    for idx in rots:
                        acc = acc + _rot16(acc, idx)
                    accs[j // 16] = jnp.where(lane == (j % 16), acc,
                                              accs[j // 16])
            for tt in range(3):
                scores_v[pl.ds(i * 48 + tt * 16, 16)] = accs[tt]
            return carry2

        lax.fori_loop(0, CB // 8, blk_body, 0)
        pltpu.sync_copy(
            scores_v,
            scores_hbm.at[pl.ds(pl.multiple_of(b0 * S, VROWS), VROWS)])

    stage_issue(0, bufs[0])

    def super_body(k, carry):
        stage_issue(2 * k + 1, bufs[1])
        wait_buf(bufs[0])
        compute(2 * k, bufs[0])

        @pl.when(k < NCHUNK // 2 - 1)
        def _():
            stage_issue(2 * k + 2, bufs[0])

        wait_buf(bufs[1])
        compute(2 * k + 1, bufs[1])
        return carry

    lax.fori_loop(0, NCHUNK // 2, super_body, 0)


def _sc_scores(word_pk, doc_pk, label_pk, u_q, v_q, v_p):
    k = functools.partial(
        pl.kernel,
        mesh=plsc.VectorSubcoreMesh(core_axis_name="c", subcore_axis_name="s"),
        out_type=jax.ShapeDtypeStruct((2 * B * S,), jnp.float32),
        scratch_types=[
            pltpu.VMEM((CB,), jnp.int32),
            pltpu.VMEM((VROWS,), jnp.int32),
            pltpu.VMEM((VROWS,), jnp.float32),
            pltpu.VMEM((CB, 2 * D), jnp.float32),
            pltpu.VMEM((VROWS, 2 * D), jnp.float32),
            pltpu.VMEM((CB,), jnp.int32),
            pltpu.VMEM((VROWS,), jnp.int32),
            pltpu.VMEM((VROWS,), jnp.float32),
            pltpu.VMEM((CB, 2 * D), jnp.float32),
            pltpu.VMEM((VROWS, 2 * D), jnp.float32),
            pltpu.VMEM((VROWS,), jnp.float32),
            pltpu.SemaphoreType.DMA,
            pltpu.SemaphoreType.DMA,
        ],
        compiler_params=pltpu.CompilerParams(use_tc_tiling_on_sc=False),
    )(_sc_scores_kernel)
    return k(word_pk, doc_pk, label_pk, u_q, v_q, v_p)


def _tr_kernel(src_ref, dst_ref):
    x = src_ref[...]
    dst_ref[:, 0:D] = x[:, 0:HALF].T
    dst_ref[:, D:2 * D] = x[:, HALF:TBLK].T


def _transpose_pack(table_t, n_rows):
    grid = (n_rows + TBLK - 1) // TBLK
    return pl.pallas_call(
        _tr_kernel,
        grid=(grid,),
        in_specs=[pl.BlockSpec((D, TBLK), lambda i: (0, i))],
        out_specs=pl.BlockSpec((HALF, 2 * D), lambda i: (i, 0)),
        out_shape=jax.ShapeDtypeStruct((grid * HALF, 2 * D), jnp.float32),
        compiler_params=pltpu.CompilerParams(
            vmem_limit_bytes=100 * 1024 * 1024),
    )(table_t)


def _pack_ids(ids):
    return ((ids >> 15) << 14) | (ids & (HALF - 1)), (ids >> 14) & 1


def _tr_dup_kernel(src_ref, dst_ref):
    x = src_ref[...].T
    dst_ref[:, 0:D] = x
    dst_ref[:, D:2 * D] = x


def _transpose_dup(table_t, n_rows):
    grid = (n_rows + TBLK - 1) // TBLK
    return pl.pallas_call(
        _tr_dup_kernel,
        grid=(grid,),
        in_specs=[pl.BlockSpec((D, TBLK), lambda i: (0, i))],
        out_specs=pl.BlockSpec((TBLK, 2 * D), lambda i: (i, 0)),
        out_shape=jax.ShapeDtypeStruct((grid * TBLK, 2 * D), jnp.float32),
        compiler_params=pltpu.CompilerParams(
            vmem_limit_bytes=100 * 1024 * 1024),
    )(table_t)


def _loss_kernel(scores_ref, out_ref):
    x = scores_ref[...]
    r = lax.broadcasted_iota(jnp.int32, x.shape, 0)
    c = lax.broadcasted_iota(jnp.int32, x.shape, 1)
    s = (r * 128 + c) % S
    t = jnp.where(s == 0, -x, x)
    sp = jnp.maximum(t, 0.0) + jnp.log1p(jnp.exp(-jnp.abs(t)))
    out_ref[0, 0] = jnp.sum(sp) * (1.0 / B)


def kernel(dt, lt, word_embed, doc_embed, label_embed, noise_a, noise_b):
    u_ids = jnp.concatenate([dt[:, 0], lt[:, 0]]).astype(jnp.int32)
    v_ids = jnp.concatenate([
        jnp.concatenate([dt[:, 1:2], noise_a], axis=1),
        jnp.concatenate([lt[:, 1:2], noise_b], axis=1)], axis=0)
    v_ids = v_ids.astype(jnp.int32).reshape(2 * B * S)
    v_q, v_p = _pack_ids(v_ids)
    v_p = v_p.astype(jnp.float32)
    word_pk = _transpose_pack(word_embed.T, 1000000)
    doc_pk = _transpose_dup(doc_embed.T, 100000)
    label_pk = _transpose_dup(label_embed.T, 1000)
    scores = _sc_scores(word_pk, doc_pk, label_pk, u_ids, v_q, v_p)
    loss2d = pl.pallas_call(
        _loss_kernel,
        out_shape=jax.ShapeDtypeStruct((1, 1), jnp.float32),
        out_specs=pl.BlockSpec(memory_space=pltpu.SMEM),
    )(scores.reshape(2 * B * S // 128, 128))
    return loss2d.reshape(())


# speedup vs baseline: 1.2038x; 1.0193x over previous
import functools

import jax
import jax.numpy as jnp
from jax import lax
from jax.experimental import pallas as pl
from jax.experimental.pallas import tpu as pltpu
from jax.experimental.pallas import tpu_sc as plsc

B = 16384
S = 6
D = 64
NW = 1000000
NC = 2
NS = 16
NSUB = NC * NS
BPT = (2 * B) // NSUB
CB = 64
NCHUNK = BPT // CB
VROWS = CB * S

TBLK = 32768
HALF = TBLK // 2

_GDN = lax.GatherDimensionNumbers(
    offset_dims=(), collapsed_slice_dims=(0,), start_index_map=(0,))


def _rot16(x, idx):
    return lax.gather(x, idx[:, None], dimension_numbers=_GDN,
                      slice_sizes=(1,),
                      mode=lax.GatherScatterMode.PROMISE_IN_BOUNDS)


def _sc_scores_kernel(word_hbm, doc_hbm, label_hbm, uq_hbm, vq_hbm,
                      vp_hbm, scores_hbm,
                      u_idx0, v_idx0, vpar0, rows_u0, rows_v0,
                      u_idx1, v_idx1, vpar1, rows_u1, rows_v1,
                      scores_v, sem0, sem1):
    wid = lax.axis_index("s") * NC + lax.axis_index("c")
    br = wid % 2
    t = wid // 2
    lane = lax.iota(jnp.int32, 16)
    rots = [(lane + (1 << k)) & 15 for k in range(4)]
    bufs = [(u_idx0, v_idx0, vpar0, rows_u0, rows_v0, sem0),
            (u_idx1, v_idx1, vpar1, rows_u1, rows_v1, sem1)]

    def stage_issue(c, buf):
        u_idx, v_idx, v_par, rows_u, rows_v, sem = buf
        b0 = br * B + t * BPT + c * CB
        pltpu.sync_copy(uq_hbm.at[pl.ds(pl.multiple_of(b0, CB), CB)], u_idx)
        pltpu.sync_copy(
            vq_hbm.at[pl.ds(pl.multiple_of(b0 * S, VROWS), VROWS)], v_idx)
        pltpu.sync_copy(
            vp_hbm.at[pl.ds(pl.multiple_of(b0 * S, VROWS), VROWS)], v_par)

        @pl.when(br == 0)
        def _():
            pltpu.async_copy(doc_hbm.at[u_idx], rows_u, sem)

        @pl.when(br == 1)
        def _():
            pltpu.async_copy(label_hbm.at[u_idx], rows_u, sem)

        for k in range(3):
            pltpu.async_copy(word_hbm.at[v_idx.at[pl.ds(k * 128, 128)]],
                             rows_v.at[pl.ds(k * 128, 128)], sem)

    def wait_buf(buf):
        u_idx, v_idx, v_par, rows_u, rows_v, sem = buf
        pltpu.make_async_copy(doc_hbm.at[u_idx], rows_u, sem).wait()
        for k in range(3):
            pltpu.make_async_copy(word_hbm.at[v_idx.at[pl.ds(k * 128, 128)]],
                                  rows_v.at[pl.ds(k * 128, 128)], sem).wait()

    def compute(c, buf):
        u_idx, v_idx, v_par, rows_u, rows_v, sem = buf
        b0 = br * B + t * BPT + c * CB

        def blk_body(i, carry2):
            accs = [jnp.zeros((16,), jnp.float32) for _ in range(3)]
            pv = [v_par[pl.ds(i * 48 + 16 * tt, 16)] for tt in range(3)]
            for bb in range(8):
                b_loc = i * 8 + bb
                u = [rows_u[b_loc, pl.ds(16 * k, 16)] for k in range(4)]
                for s in range(S):
                    r = b_loc * S + s
                    j = bb * S + s
                    alo = u[0] * rows_v[r, pl.ds(0, 16)]
                    ahi = u[0] * rows_v[r, pl.ds(64, 16)]
                    alo += u[1] * rows_v[r, pl.ds(16, 16)]
                    ahi += u[1] * rows_v[r, pl.ds(80, 16)]
                    alo += u[2] * rows_v[r, pl.ds(32, 16)]
                    ahi += u[2] * rows_v[r, pl.ds(96, 16)]
                    alo += u[3] * rows_v[r, pl.ds(48, 16)]
                    ahi += u[3] * rows_v[r, pl.ds(112, 16)]
                    parf = _rot16(pv[j // 16],
                                  jnp.full((16,), j % 16, jnp.int32))
                    acc = alo + parf * (ahi - alo)
                    for idx in rots:
                        acc = acc + _rot16(acc, idx)
                    accs[j // 16] = jnp.where(lane == (j % 16), acc,
                                              accs[j // 16])
            for tt in range(3):
                scores_v[pl.ds(i * 48 + tt * 16, 16)] = accs[tt]
            return carry2

        lax.fori_loop(0, CB // 8, blk_body, 0)
        pltpu.sync_copy(
            scores_v,
            scores_hbm.at[pl.ds(pl.multiple_of(b0 * S, VROWS), VROWS)])

    stage_issue(0, bufs[0])

    def super_body(k, carry):
        stage_issue(2 * k + 1, bufs[1])
        wait_buf(bufs[0])
        compute(2 * k, bufs[0])

        @pl.when(k < NCHUNK // 2 - 1)
        def _():
            stage_issue(2 * k + 2, bufs[0])

        wait_buf(bufs[1])
        compute(2 * k + 1, bufs[1])
        return carry

    lax.fori_loop(0, NCHUNK // 2, super_body, 0)


def _sc_scores(word_pk, doc_pk, label_pk, u_q, v_q, v_p):
    k = functools.partial(
        pl.kernel,
        mesh=plsc.VectorSubcoreMesh(core_axis_name="c", subcore_axis_name="s"),
        out_type=jax.ShapeDtypeStruct((2 * B * S,), jnp.float32),
        scratch_types=[
            pltpu.VMEM((CB,), jnp.int32),
            pltpu.VMEM((VROWS,), jnp.int32),
            pltpu.VMEM((VROWS,), jnp.float32),
            pltpu.VMEM((CB, D), jnp.float32),
            pltpu.VMEM((VROWS, 2 * D), jnp.float32),
            pltpu.VMEM((CB,), jnp.int32),
            pltpu.VMEM((VROWS,), jnp.int32),
            pltpu.VMEM((VROWS,), jnp.float32),
            pltpu.VMEM((CB, D), jnp.float32),
            pltpu.VMEM((VROWS, 2 * D), jnp.float32),
            pltpu.VMEM((VROWS,), jnp.float32),
            pltpu.SemaphoreType.DMA,
            pltpu.SemaphoreType.DMA,
        ],
        compiler_params=pltpu.CompilerParams(use_tc_tiling_on_sc=False),
    )(_sc_scores_kernel)
    return k(word_pk, doc_pk, label_pk, u_q, v_q, v_p)


def _tr_kernel(src_ref, dst_ref):
    x = src_ref[...]
    dst_ref[:, 0:D] = x[:, 0:HALF].T
    dst_ref[:, D:2 * D] = x[:, HALF:TBLK].T


def _transpose_pack(table_t, n_rows):
    grid = (n_rows + TBLK - 1) // TBLK
    return pl.pallas_call(
        _tr_kernel,
        grid=(grid,),
        in_specs=[pl.BlockSpec((D, TBLK), lambda i: (0, i))],
        out_specs=pl.BlockSpec((HALF, 2 * D), lambda i: (i, 0)),
        out_shape=jax.ShapeDtypeStruct((grid * HALF, 2 * D), jnp.float32),
        compiler_params=pltpu.CompilerParams(
            vmem_limit_bytes=100 * 1024 * 1024),
    )(table_t)


def _pack_ids(ids):
    return ((ids >> 15) << 14) | (ids & (HALF - 1)), (ids >> 14) & 1


def _tr_dup_kernel(src_ref, dst_ref):
    x = src_ref[...].T
    dst_ref[:, 0:D] = x
    dst_ref[:, D:2 * D] = x


def _transpose_dup(table_t, n_rows):
    grid = (n_rows + TBLK - 1) // TBLK
    return pl.pallas_call(
        _tr_dup_kernel,
        grid=(grid,),
        in_specs=[pl.BlockSpec((D, TBLK), lambda i: (0, i))],
        out_specs=pl.BlockSpec((TBLK, 2 * D), lambda i: (i, 0)),
        out_shape=jax.ShapeDtypeStruct((grid * TBLK, 2 * D), jnp.float32),
        compiler_params=pltpu.CompilerParams(
            vmem_limit_bytes=100 * 1024 * 1024),
    )(table_t)


def _loss_kernel(scores_ref, out_ref):
    x = scores_ref[...]
    r = lax.broadcasted_iota(jnp.int32, x.shape, 0)
    c = lax.broadcasted_iota(jnp.int32, x.shape, 1)
    s = (r * 128 + c) % S
    t = jnp.where(s == 0, -x, x)
    sp = jnp.maximum(t, 0.0) + jnp.log1p(jnp.exp(-jnp.abs(t)))
    out_ref[0, 0] = jnp.sum(sp) * (1.0 / B)


def kernel(dt, lt, word_embed, doc_embed, label_embed, noise_a, noise_b):
    u_ids = jnp.concatenate([dt[:, 0], lt[:, 0]]).astype(jnp.int32)
    v_ids = jnp.concatenate([
        jnp.concatenate([dt[:, 1:2], noise_a], axis=1),
        jnp.concatenate([lt[:, 1:2], noise_b], axis=1)], axis=0)
    v_ids = v_ids.astype(jnp.int32).reshape(2 * B * S)
    v_q, v_p = _pack_ids(v_ids)
    v_p = v_p.astype(jnp.float32)
    word_pk = _transpose_pack(word_embed.T, 1000000)
    scores = _sc_scores(word_pk, doc_embed, label_embed, u_ids, v_q, v_p)
    loss2d = pl.pallas_call(
        _loss_kernel,
        out_shape=jax.ShapeDtypeStruct((1, 1), jnp.float32),
        out_specs=pl.BlockSpec(memory_space=pltpu.SMEM),
    )(scores.reshape(2 * B * S // 128, 128))
    return loss2d.reshape(())
